# baseline (speedup 1.0000x reference)
import functools

import jax
import jax.numpy as jnp
from jax import lax
from jax.experimental import pallas as pl
from jax.experimental.pallas import tpu as pltpu
from jax.experimental.pallas import tpu_sc as plsc

_D = 64
_SCALE = 8.0

_NC = 2
_NS = 16
_NW = _NC * _NS

_BLK = 128
_RING = 4


@functools.partial(jax.jit, static_argnames=("n_t", "n_b"))
def _embed_lookup(table, x_t, n_t, n_b):
    n_bc = n_b // _BLK
    assert n_bc == _NW
    mesh = plsc.VectorSubcoreMesh(core_axis_name="c", subcore_axis_name="s")

    scratch = [pltpu.VMEM((n_t, _BLK), jnp.int32)]
    scratch += [pltpu.VMEM((_BLK, _D), jnp.float32) for _ in range(_RING)]
    scratch += [pltpu.VMEM((_D // 8, 8, _BLK), jnp.float32)
                for _ in range(_RING)]
    scratch += [pltpu.SemaphoreType.DMA for _ in range(2 * _RING)]

    @functools.partial(
        pl.kernel,
        out_type=jax.ShapeDtypeStruct((n_t, _D // 8, _NW, 8, _BLK),
                                      jnp.float32),
        mesh=mesh,
        scratch_types=scratch,
        compiler_params=pltpu.CompilerParams(
            use_tc_tiling_on_sc=False, needs_layout_passes=False),
    )
    def k(table_hbm, xt_hbm, out_hbm, idx_all, *bufs):
        rows = bufs[0:_RING]
        trans = bufs[_RING:2 * _RING]
        gsem = bufs[2 * _RING:3 * _RING]
        wsem = bufs[3 * _RING:4 * _RING]
        bc = lax.axis_index("s") * _NC + lax.axis_index("c")

        pltpu.sync_copy(xt_hbm.at[:, pl.ds(bc * _BLK, _BLK)], idx_all)

        def fire_gather(q, t):
            pltpu.async_copy(table_hbm.at[idx_all.at[t]], rows[q], gsem[q])

        for q in range(_RING):
            fire_gather(q, q)

        iota16 = lax.iota(jnp.int32, 16)
        rowv = [iota16 + c16 * 16 for c16 in range(_BLK // 16)]

        def transpose_scale(q):
            def dbody(d, carry):
                colv = jnp.broadcast_to(d, (16,)).astype(jnp.int32)
                dr = d // 8
                rbase = d % 8
                for c16 in range(_BLK // 16):
                    vals = plsc.load_gather(rows[q], [rowv[c16], colv])
                    trans[q][dr, rbase, pl.ds(c16 * 16, 16)] = vals * _SCALE
                return carry
            lax.fori_loop(0, _D, dbody, 0)

        def outer(h, carry):
            for qi in range(_RING):
                t = h * _RING + qi
                pltpu.make_async_copy(
                    table_hbm.at[idx_all.at[t]], rows[qi], gsem[qi]).wait()

                @pl.when(t >= _RING)
                def _():
                    pltpu.make_async_copy(
                        trans[qi], out_hbm.at[t, :, bc, :, :],
                        wsem[qi]).wait()

                transpose_scale(qi)
                pltpu.async_copy(
                    trans[qi], out_hbm.at[t, :, bc, :, :], wsem[qi])

                @pl.when(t + _RING < n_t)
                def _():
                    fire_gather(qi, t + _RING)
            return carry

        lax.fori_loop(0, n_t // _RING, outer, 0)

        for q in range(_RING):
            pltpu.make_async_copy(
                trans[q], out_hbm.at[0, :, bc, :, :], wsem[q]).wait()

    return k(table, x_t)


def kernel(x, table):
    n_b, n_t = x.shape
    d = table.shape[1]
    x_t = x.T.astype(jnp.int32)
    o5 = _embed_lookup(table, x_t, n_t, n_b)
    return o5.transpose(2, 4, 0, 1, 3).reshape(n_b, n_t, d)
